# Initial kernel scaffold; baseline (speedup 1.0000x reference)
import jax
import jax.numpy as jnp
from jax.experimental import pallas as pl


def kernel(input_s, output_s, input_r, output_r, label):
    raise NotImplementedError("write your pallas kernel here")


# speedup vs baseline: 2.8784x; 2.8784x over previous
import jax
import jax.numpy as jnp
from jax.experimental import pallas as pl
from jax.experimental.pallas import tpu as pltpu

_TB = 256


def _ce_kernel(s_ref, x_ref, loss_ref, acc_ref, num_ref, den_ref):
    b = pl.program_id(0)
    t = pl.program_id(1)
    nb = pl.num_programs(0)
    nt = pl.num_programs(1)
    tb = x_ref.shape[1]
    v = x_ref.shape[2]

    x = x_ref[0]
    srow = s_ref[0, 0]
    length = jnp.sum((srow != 0).astype(jnp.int32))
    tgt = s_ref[0, 0, pl.ds(t * tb, tb)]

    m = jnp.max(x, axis=1, keepdims=True)
    ssum = jnp.sum(jnp.exp(x - m), axis=1, keepdims=True)
    lse = m + jnp.log(ssum)

    lane = jax.lax.broadcasted_iota(jnp.int32, (tb, v), 1)
    tgt2 = tgt[:, None]
    tgt_val = jnp.sum(jnp.where(lane == tgt2, x, 0.0), axis=1, keepdims=True)
    amax = jnp.min(jnp.where(x == m, lane, v), axis=1, keepdims=True)

    rows = jax.lax.broadcasted_iota(jnp.int32, (tb, 1), 0) + t * tb
    pmask = (rows < length).astype(jnp.float32)

    loss_part = jnp.sum((lse - tgt_val) * pmask)
    corr_part = jnp.sum((amax == tgt2).astype(jnp.float32) * pmask)
    nvalid = jnp.sum(pmask)

    @pl.when(t == 0)
    def _():
        loss_ref[0, 0, :] = jnp.zeros((128,), jnp.float32)

    loss_ref[0, 0, :] += jnp.full((128,), loss_part, jnp.float32)

    @pl.when(jnp.logical_and(b == 0, t == 0))
    def _():
        num_ref[0] = 0.0
        den_ref[0] = 0.0

    num_ref[0] += corr_part
    den_ref[0] += nvalid

    @pl.when(jnp.logical_and(b == nb - 1, t == nt - 1))
    def _():
        acc_ref[0, :] = jnp.full((128,), num_ref[0] / den_ref[0], jnp.float32)


def kernel(input_s, output_s, input_r, output_r, label):
    B, T = input_s.shape
    V = output_r.shape[-1]
    nt = T // _TB

    loss2d, acc2d = pl.pallas_call(
        _ce_kernel,
        grid=(B, nt),
        in_specs=[
            pl.BlockSpec((1, 1, T), lambda b, t: (b, 0, 0)),
            pl.BlockSpec((1, _TB, V), lambda b, t: (b, t, 0)),
        ],
        out_specs=[
            pl.BlockSpec((1, 1, 128), lambda b, t: (b, 0, 0)),
            pl.BlockSpec((1, 128), lambda b, t: (0, 0)),
        ],
        out_shape=[
            jax.ShapeDtypeStruct((B, 1, 128), jnp.float32),
            jax.ShapeDtypeStruct((1, 128), jnp.float32),
        ],
        scratch_shapes=[
            pltpu.SMEM((1,), jnp.float32),
            pltpu.SMEM((1,), jnp.float32),
        ],
    )(input_s.reshape(B, 1, T), output_r)

    loss = loss2d[:, 0, 0]
    acc = acc2d[0, 0]
    return (loss, acc)
